# baseline (speedup 1.0000x reference)
import functools

import jax
import jax.numpy as jnp
from jax import lax
from jax.experimental import pallas as pl
from jax.experimental.pallas import tpu as pltpu
from jax.experimental.pallas import tpu_sc as plsc

_TN = 16000
_N = 320000
_TILES = _N // _TN


def _fused_kernel(p_ref, w1_ref, w2r_ref, g1_ref, be1_ref,
                  sum1_ref, sq1_ref, sum2_ref, sq2_ref, z2_ref,
                  scale_ref, shift_ref):
    i = pl.program_id(0)
    z1 = jnp.dot(p_ref[...], w1_ref[...], preferred_element_type=jnp.float32)

    @pl.when(i == 0)
    def _init_a():
        sum1_ref[...] = jnp.zeros_like(sum1_ref)
        sq1_ref[...] = jnp.zeros_like(sq1_ref)

    @pl.when(i < _TILES)
    def _phase_a():
        s = jnp.sum(z1, axis=0, keepdims=True)
        q = jnp.sum(z1 * z1, axis=0, keepdims=True)
        sum1_ref[...] += jnp.broadcast_to(s, sum1_ref.shape)
        sq1_ref[...] += jnp.broadcast_to(q, sq1_ref.shape)

    @pl.when(i == _TILES)
    def _prep():
        inv_n = 1.0 / jnp.float32(_N)
        mean1 = sum1_ref[0:1, :] * inv_n
        var1 = sq1_ref[0:1, :] * inv_n - mean1 * mean1
        inv1 = g1_ref[...] * lax.rsqrt(var1 + 1e-5)
        scale_ref[...] = inv1.T
        shift_ref[...] = (be1_ref[...] - mean1 * inv1).T
        sum2_ref[...] = jnp.zeros_like(sum2_ref)
        sq2_ref[...] = jnp.zeros_like(sq2_ref)

    @pl.when(i >= _TILES)
    def _phase_b():
        z1t = z1.T
        h = jnp.maximum(z1t * scale_ref[...] + shift_ref[...], 0.0)
        z2 = jnp.dot(w2r_ref[...], h,
                     preferred_element_type=jnp.float32)
        z2_ref[pl.ds((i - _TILES) * _TN, _TN)] = z2.reshape((_TN,))
        sum2_ref[...] += jnp.full(sum2_ref.shape, jnp.sum(z2), jnp.float32)
        sq2_ref[...] += jnp.full(sq2_ref.shape, jnp.sum(z2 * z2), jnp.float32)


def _make_seg_max(n):
    info = plsc.get_sparse_core_info()
    nw = info.num_cores * info.num_subcores
    chunk = n // nw
    mesh = plsc.VectorSubcoreMesh(core_axis_name="c", subcore_axis_name="s")

    @functools.partial(
        pl.kernel,
        out_type=jax.ShapeDtypeStruct((nw, 16), jnp.float32),
        mesh=mesh,
        scratch_types=[
            pltpu.VMEM((chunk + 16,), jnp.int32),
            pltpu.VMEM((chunk,), jnp.float32),
            pltpu.VMEM((16,), jnp.float32),
            pltpu.SemaphoreType.DMA,
            pltpu.SemaphoreType.DMA,
        ],
    )
    def seg_max(idx_hbm, z2_hbm, out_hbm, idx_v, z2_v, res_v, sem1, sem2):
        wid = lax.axis_index("s") * info.num_cores + lax.axis_index("c")
        base = wid * chunk
        cp1 = pltpu.async_copy(idx_hbm.at[pl.ds(base, chunk + 16)], idx_v, sem1)
        cp2 = pltpu.async_copy(z2_hbm.at[pl.ds(base, chunk)], z2_v, sem2)
        cp1.wait()
        cp2.wait()

        unroll = 25

        def body(k, m):
            for j in range(unroll):
                o = (k * unroll + j) * 16
                a = idx_v[pl.ds(o, 16)]
                b = idx_v[pl.ds(o + 1, 16)]
                z = z2_v[pl.ds(o, 16)]
                m = jnp.maximum(m, jnp.where(a != b, z, -jnp.inf))
            return m

        m = lax.fori_loop(0, chunk // (16 * unroll), body,
                          jnp.full((16,), -jnp.inf, jnp.float32))
        res_v[...] = m
        pltpu.sync_copy(res_v, out_hbm.at[wid])

    return seg_max


@jax.jit
def kernel(points, inverse_indices, W1, b1, gamma1, beta1,
           W2, b2, gamma2, beta2):
    n, d_in = points.shape
    f1 = W1.shape[1]
    eps = 1e-5

    w2r = W2[:, 0].reshape(1, f1)
    g1 = gamma1.reshape(1, f1)
    be1 = beta1.reshape(1, f1)

    sum1, sq1, sum2, sq2, z2_flat = pl.pallas_call(
        _fused_kernel,
        grid=(2 * _TILES,),
        in_specs=[
            pl.BlockSpec((_TN, d_in), lambda i: (i % _TILES, 0)),
            pl.BlockSpec((d_in, f1), lambda i: (0, 0)),
            pl.BlockSpec((1, f1), lambda i: (0, 0)),
            pl.BlockSpec((1, f1), lambda i: (0, 0)),
            pl.BlockSpec((1, f1), lambda i: (0, 0)),
        ],
        scratch_shapes=[
            pltpu.VMEM((f1, 1), jnp.float32),
            pltpu.VMEM((f1, 1), jnp.float32),
        ],
        out_specs=[
            pl.BlockSpec((8, f1), lambda i: (0, 0)),
            pl.BlockSpec((8, f1), lambda i: (0, 0)),
            pl.BlockSpec((8, 128), lambda i: (0, 0)),
            pl.BlockSpec((8, 128), lambda i: (0, 0)),
            pl.BlockSpec((_N,), lambda i: (0,)),
        ],
        out_shape=[
            jax.ShapeDtypeStruct((8, f1), jnp.float32),
            jax.ShapeDtypeStruct((8, f1), jnp.float32),
            jax.ShapeDtypeStruct((8, 128), jnp.float32),
            jax.ShapeDtypeStruct((8, 128), jnp.float32),
            jax.ShapeDtypeStruct((n,), jnp.float32),
        ],
    )(points, W1, w2r, g1, be1)

    idx_ext = jnp.concatenate(
        [inverse_indices, jnp.full((16,), -1, jnp.int32)])
    partials = _make_seg_max(n)(idx_ext, z2_flat)

    mean2 = sum2[0, 0] / n
    var2 = sq2[0, 0] / n - mean2 * mean2
    m = jnp.max(partials)
    out = (m - mean2) / jnp.sqrt(var2 + eps) * gamma2[0] + beta2[0]
    return jnp.maximum(out, 0.0)


# speedup vs baseline: 1.1577x; 1.1577x over previous
import functools

import jax
import jax.numpy as jnp
from jax import lax
from jax.experimental import pallas as pl
from jax.experimental.pallas import tpu as pltpu
from jax.experimental.pallas import tpu_sc as plsc

_TN = 16000
_N = 320000


def _stats1_kernel(p_ref, w1_ref, sum_ref, sq_ref, z1t_ref):
    z1 = jnp.dot(p_ref[...], w1_ref[...], preferred_element_type=jnp.float32)
    z1t_ref[0] = z1.astype(jnp.bfloat16).T
    s = jnp.sum(z1, axis=0, keepdims=True)
    q = jnp.sum(z1 * z1, axis=0, keepdims=True)

    @pl.when(pl.program_id(0) == 0)
    def _init():
        sum_ref[...] = jnp.broadcast_to(s, sum_ref.shape)
        sq_ref[...] = jnp.broadcast_to(q, sq_ref.shape)

    @pl.when(pl.program_id(0) != 0)
    def _acc():
        sum_ref[...] += jnp.broadcast_to(s, sum_ref.shape)
        sq_ref[...] += jnp.broadcast_to(q, sq_ref.shape)


def _pass2_kernel(z1t_ref, w2r_ref, sum1_ref, sq1_ref, g1_ref, be1_ref,
                  sum_ref, sq_ref, z2_ref, scale_ref, shift_ref):
    @pl.when(pl.program_id(0) == 0)
    def _prep():
        inv_n = 1.0 / jnp.float32(_N)
        mean1 = sum1_ref[0:1, :] * inv_n
        var1 = sq1_ref[0:1, :] * inv_n - mean1 * mean1
        inv1 = g1_ref[...] * jax.lax.rsqrt(var1 + 1e-5)
        scale_ref[...] = inv1.T
        shift_ref[...] = (be1_ref[...] - mean1 * inv1).T

    z1t = z1t_ref[0].astype(jnp.float32)
    h = jnp.maximum(z1t * scale_ref[...] + shift_ref[...], 0.0)
    z2 = jnp.dot(w2r_ref[...], h, preferred_element_type=jnp.float32)
    z2_ref[pl.ds(pl.program_id(0) * _TN, _TN)] = z2.reshape((_TN,))
    s = jnp.sum(z2)
    q = jnp.sum(z2 * z2)

    @pl.when(pl.program_id(0) == 0)
    def _init():
        sum_ref[...] = jnp.full(sum_ref.shape, s, jnp.float32)
        sq_ref[...] = jnp.full(sq_ref.shape, q, jnp.float32)

    @pl.when(pl.program_id(0) != 0)
    def _acc():
        sum_ref[...] += jnp.full(sum_ref.shape, s, jnp.float32)
        sq_ref[...] += jnp.full(sq_ref.shape, q, jnp.float32)


def _make_seg_max(n):
    info = plsc.get_sparse_core_info()
    nw = info.num_cores * info.num_subcores
    chunk = n // nw
    mesh = plsc.VectorSubcoreMesh(core_axis_name="c", subcore_axis_name="s")

    @functools.partial(
        pl.kernel,
        out_type=jax.ShapeDtypeStruct((nw, 16), jnp.float32),
        mesh=mesh,
        scratch_types=[
            pltpu.VMEM((chunk + 16,), jnp.int32),
            pltpu.VMEM((chunk,), jnp.float32),
            pltpu.VMEM((16,), jnp.float32),
            pltpu.SemaphoreType.DMA,
            pltpu.SemaphoreType.DMA,
        ],
    )
    def seg_max(idx_hbm, z2_hbm, out_hbm, idx_v, z2_v, res_v, sem1, sem2):
        wid = lax.axis_index("s") * info.num_cores + lax.axis_index("c")
        base = wid * chunk
        cp1 = pltpu.async_copy(idx_hbm.at[pl.ds(base, chunk + 16)], idx_v, sem1)
        cp2 = pltpu.async_copy(z2_hbm.at[pl.ds(base, chunk)], z2_v, sem2)
        cp1.wait()
        cp2.wait()

        unroll = 25

        def body(k, m):
            for j in range(unroll):
                o = (k * unroll + j) * 16
                a = idx_v[pl.ds(o, 16)]
                b = idx_v[pl.ds(o + 1, 16)]
                z = z2_v[pl.ds(o, 16)]
                m = jnp.maximum(m, jnp.where(a != b, z, -jnp.inf))
            return m

        m = lax.fori_loop(0, chunk // (16 * unroll), body,
                          jnp.full((16,), -jnp.inf, jnp.float32))
        res_v[...] = m
        pltpu.sync_copy(res_v, out_hbm.at[wid])

    return seg_max


@jax.jit
def kernel(points, inverse_indices, W1, b1, gamma1, beta1,
           W2, b2, gamma2, beta2):
    n, d_in = points.shape
    f1 = W1.shape[1]
    tiles = n // _TN
    eps = 1e-5

    sum1, sq1, z1t = pl.pallas_call(
        _stats1_kernel,
        grid=(tiles,),
        in_specs=[
            pl.BlockSpec((_TN, d_in), lambda i: (i, 0)),
            pl.BlockSpec((d_in, f1), lambda i: (0, 0)),
        ],
        out_specs=[
            pl.BlockSpec((8, f1), lambda i: (0, 0)),
            pl.BlockSpec((8, f1), lambda i: (0, 0)),
            pl.BlockSpec((1, f1, _TN), lambda i: (i, 0, 0)),
        ],
        out_shape=[
            jax.ShapeDtypeStruct((8, f1), jnp.float32),
            jax.ShapeDtypeStruct((8, f1), jnp.float32),
            jax.ShapeDtypeStruct((tiles, f1, _TN), jnp.bfloat16),
        ],
    )(points, W1)

    w2r = W2[:, 0].reshape(1, f1)
    g1 = gamma1.reshape(1, f1)
    be1 = beta1.reshape(1, f1)

    sum2, sq2, z2_flat = pl.pallas_call(
        _pass2_kernel,
        grid=(tiles,),
        in_specs=[
            pl.BlockSpec((1, f1, _TN), lambda i: (i, 0, 0)),
            pl.BlockSpec((1, f1), lambda i: (0, 0)),
            pl.BlockSpec((8, f1), lambda i: (0, 0)),
            pl.BlockSpec((8, f1), lambda i: (0, 0)),
            pl.BlockSpec((1, f1), lambda i: (0, 0)),
            pl.BlockSpec((1, f1), lambda i: (0, 0)),
        ],
        scratch_shapes=[
            pltpu.VMEM((f1, 1), jnp.float32),
            pltpu.VMEM((f1, 1), jnp.float32),
        ],
        out_specs=[
            pl.BlockSpec((8, 128), lambda i: (0, 0)),
            pl.BlockSpec((8, 128), lambda i: (0, 0)),
            pl.BlockSpec((_N,), lambda i: (0,)),
        ],
        out_shape=[
            jax.ShapeDtypeStruct((8, 128), jnp.float32),
            jax.ShapeDtypeStruct((8, 128), jnp.float32),
            jax.ShapeDtypeStruct((n,), jnp.float32),
        ],
    )(z1t, w2r, sum1, sq1, g1, be1)

    idx_ext = jnp.concatenate(
        [inverse_indices, jnp.full((16,), -1, jnp.int32)])
    partials = _make_seg_max(n)(idx_ext, z2_flat)

    mean2 = sum2[0, 0] / n
    var2 = sq2[0, 0] / n - mean2 * mean2
    m = jnp.max(partials)
    out = (m - mean2) / jnp.sqrt(var2 + eps) * gamma2[0] + beta2[0]
    return jnp.maximum(out, 0.0)
